# baseline (device time: 22458 ns/iter reference)
import jax
import jax.numpy as jnp
from jax import lax
from jax.experimental import pallas as pl
from jax.experimental.pallas import tpu as pltpu

N_DEV = 32


def kernel(t, W):
    m, k = t.shape
    n = W.shape[1]
    pc = m // 8

    def body(t_ref, w_ref, out_ref, tb_ref, s1_ref, qp_ref, ago_ref,
             s1_ssem, s1_rsem, s2_ssem, s2_rsem, s3_ssem, s3_rsem):
        my = lax.axis_index("i")
        z = my // 8
        o = jnp.mod(my, 8)
        y = o // 2
        x = jnp.mod(o + y, 2)
        r = x * 4 + z

        def id_of(xx, yy, zz):
            return zz * 8 + 2 * yy + jnp.mod(xx + yy, 2)

        p_grid = []
        for kk in range(1, 8):
            rp = jnp.mod(r + kk, 8)
            p_grid.append(id_of(rp // 4, y, jnp.mod(rp, 4)))
        p_line = [id_of(x, jnp.mod(y + kk, 4), z) for kk in (1, 2, 3)]

        tb_ref[:, :] = t_ref[:, :].astype(jnp.bfloat16)

        barrier = pltpu.get_barrier_semaphore()
        for p in p_grid + p_line:
            pl.semaphore_signal(
                barrier, inc=1,
                device_id=(p,), device_id_type=pl.DeviceIdType.MESH,
            )
        pl.semaphore_wait(barrier, 10)

        sends = []

        for kk in range(1, 8):
            rp = jnp.mod(r + kk, 8)
            rdma = pltpu.make_async_remote_copy(
                src_ref=tb_ref.at[pl.ds(rp * pc, pc)],
                dst_ref=s1_ref.at[8 - kk],
                send_sem=s1_ssem.at[kk],
                recv_sem=s1_rsem.at[8 - kk],
                device_id=(p_grid[kk - 1],),
                device_id_type=pl.DeviceIdType.MESH,
            )
            rdma.start()
            sends.append(rdma)
        for j in range(1, 8):
            pltpu.make_async_remote_copy(
                src_ref=s1_ref.at[j], dst_ref=s1_ref.at[j],
                send_sem=s1_ssem.at[j], recv_sem=s1_rsem.at[j],
                device_id=(my,), device_id_type=pl.DeviceIdType.MESH,
            ).wait_recv()
        partial = t_ref[pl.ds(r * pc, pc), :] + jnp.sum(
            s1_ref[pl.ds(1, 7), :, :].astype(jnp.float32), axis=0
        )
        qp_ref[0, :, :] = partial.astype(jnp.bfloat16)

        for kk in range(1, 4):
            rdma = pltpu.make_async_remote_copy(
                src_ref=qp_ref.at[0],
                dst_ref=qp_ref.at[4 - kk],
                send_sem=s2_ssem.at[kk],
                recv_sem=s2_rsem.at[4 - kk],
                device_id=(p_line[kk - 1],),
                device_id_type=pl.DeviceIdType.MESH,
            )
            rdma.start()
            sends.append(rdma)
        for j in range(1, 4):
            pltpu.make_async_remote_copy(
                src_ref=qp_ref.at[j], dst_ref=qp_ref.at[j],
                send_sem=s2_ssem.at[j], recv_sem=s2_rsem.at[j],
                device_id=(my,), device_id_type=pl.DeviceIdType.MESH,
            ).wait_recv()
        q_full = partial + jnp.sum(
            qp_ref[pl.ds(1, 3), :, :].astype(jnp.float32), axis=0
        )

        ymat = jnp.dot(q_full, w_ref[:, :], preferred_element_type=jnp.float32)
        out_ref[pl.ds(r * pc, pc), :] = ymat
        ago_ref[pl.ds(r, 1), :, :] = ymat.astype(jnp.bfloat16)[None]

        for kk in range(1, 8):
            rdma = pltpu.make_async_remote_copy(
                src_ref=ago_ref.at[pl.ds(r, 1)],
                dst_ref=ago_ref.at[pl.ds(r, 1)],
                send_sem=s3_ssem.at[kk],
                recv_sem=s3_rsem.at[8 - kk],
                device_id=(p_grid[kk - 1],),
                device_id_type=pl.DeviceIdType.MESH,
            )
            rdma.start()
            sends.append(rdma)

        for rdma in sends[:10]:
            rdma.wait_send()

        for j in range(1, 8):
            rs = jnp.mod(r + j, 8)
            pltpu.make_async_remote_copy(
                src_ref=ago_ref.at[pl.ds(rs, 1)],
                dst_ref=ago_ref.at[pl.ds(rs, 1)],
                send_sem=s3_ssem.at[j], recv_sem=s3_rsem.at[j],
                device_id=(my,), device_id_type=pl.DeviceIdType.MESH,
            ).wait_recv()
            out_ref[pl.ds(rs * pc, pc), :] = (
                ago_ref[pl.ds(rs, 1), :, :].astype(jnp.float32)[0]
            )

        for rdma in sends[10:]:
            rdma.wait_send()

    return pl.pallas_call(
        body,
        out_shape=jax.ShapeDtypeStruct((m, n), jnp.float32),
        in_specs=[
            pl.BlockSpec(memory_space=pltpu.VMEM),
            pl.BlockSpec(memory_space=pltpu.VMEM),
        ],
        out_specs=pl.BlockSpec(memory_space=pltpu.VMEM),
        scratch_shapes=[
            pltpu.VMEM((m, k), jnp.bfloat16),
            pltpu.VMEM((8, pc, k), jnp.bfloat16),
            pltpu.VMEM((4, pc, k), jnp.bfloat16),
            pltpu.VMEM((8, pc, n), jnp.bfloat16),
            pltpu.SemaphoreType.DMA((8,)),
            pltpu.SemaphoreType.DMA((8,)),
            pltpu.SemaphoreType.DMA((4,)),
            pltpu.SemaphoreType.DMA((4,)),
            pltpu.SemaphoreType.DMA((8,)),
            pltpu.SemaphoreType.DMA((8,)),
        ],
        compiler_params=pltpu.CompilerParams(collective_id=0),
    )(t, W)


# device time: 21181 ns/iter; 1.0603x vs baseline; 1.0603x over previous
import jax
import jax.numpy as jnp
from jax import lax
from jax.experimental import pallas as pl
from jax.experimental.pallas import tpu as pltpu

N_DEV = 32


def kernel(t, W):
    m, k = t.shape
    n = W.shape[1]
    pc = m // 8
    ch = m // N_DEV

    def body(t_ref, w_ref, out_ref, tb_ref, s1_ref, py_ref, s2_ref, ag_ref,
             s1_ssem, s1_rsem, s2_ssem, s2_rsem,
             agA_ssem, agA_rsem, agB_ssem, agB_rsem):
        my = lax.axis_index("i")
        z = my // 8
        o = jnp.mod(my, 8)
        y = o // 2
        x = jnp.mod(o + y, 2)
        r = x * 4 + z
        g = r * 4 + y

        def id_of(xx, yy, zz):
            return zz * 8 + 2 * yy + jnp.mod(xx + yy, 2)

        p_grid = []
        for kk in range(1, 8):
            rp = jnp.mod(r + kk, 8)
            p_grid.append(id_of(rp // 4, y, jnp.mod(rp, 4)))
        p_line = [id_of(x, jnp.mod(y + kk, 4), z) for kk in (1, 2, 3)]

        tb_ref[:, :] = t_ref[:, :].astype(jnp.bfloat16)

        barrier = pltpu.get_barrier_semaphore()
        for p in p_grid + p_line:
            pl.semaphore_signal(
                barrier, inc=1,
                device_id=(p,), device_id_type=pl.DeviceIdType.MESH,
            )
        pl.semaphore_wait(barrier, 10)

        sends = []

        for kk in range(1, 8):
            rp = jnp.mod(r + kk, 8)
            rdma = pltpu.make_async_remote_copy(
                src_ref=tb_ref.at[pl.ds(rp * pc, pc)],
                dst_ref=s1_ref.at[8 - kk],
                send_sem=s1_ssem.at[kk],
                recv_sem=s1_rsem.at[8 - kk],
                device_id=(p_grid[kk - 1],),
                device_id_type=pl.DeviceIdType.MESH,
            )
            rdma.start()
            sends.append(rdma)
        for j in range(1, 8):
            pltpu.make_async_remote_copy(
                src_ref=s1_ref.at[j], dst_ref=s1_ref.at[j],
                send_sem=s1_ssem.at[j], recv_sem=s1_rsem.at[j],
                device_id=(my,), device_id_type=pl.DeviceIdType.MESH,
            ).wait_recv()
        tot = t_ref[pl.ds(r * pc, pc), :] + jnp.sum(
            s1_ref[pl.ds(1, 7), :, :].astype(jnp.float32), axis=0
        )
        py_ref[:, :] = tot.astype(jnp.bfloat16)

        for kk in range(1, 4):
            yp = jnp.mod(y + kk, 4)
            rdma = pltpu.make_async_remote_copy(
                src_ref=py_ref.at[pl.ds(yp * ch, ch)],
                dst_ref=s2_ref.at[4 - kk],
                send_sem=s2_ssem.at[kk],
                recv_sem=s2_rsem.at[4 - kk],
                device_id=(p_line[kk - 1],),
                device_id_type=pl.DeviceIdType.MESH,
            )
            rdma.start()
            sends.append(rdma)
        for j in range(1, 4):
            pltpu.make_async_remote_copy(
                src_ref=s2_ref.at[j], dst_ref=s2_ref.at[j],
                send_sem=s2_ssem.at[j], recv_sem=s2_rsem.at[j],
                device_id=(my,), device_id_type=pl.DeviceIdType.MESH,
            ).wait_recv()
        acc = py_ref[pl.ds(y * ch, ch), :].astype(jnp.float32) + jnp.sum(
            s2_ref[pl.ds(1, 3), :, :].astype(jnp.float32), axis=0
        )

        ymat = jnp.dot(acc, w_ref[:, :], preferred_element_type=jnp.float32)
        ag_ref[pl.ds(g, 1), :, :] = ymat.astype(jnp.bfloat16)[None]

        for kk in range(1, 4):
            rdma = pltpu.make_async_remote_copy(
                src_ref=ag_ref.at[pl.ds(g, 1)],
                dst_ref=ag_ref.at[pl.ds(g, 1)],
                send_sem=agA_ssem.at[kk],
                recv_sem=agA_rsem.at[4 - kk],
                device_id=(p_line[kk - 1],),
                device_id_type=pl.DeviceIdType.MESH,
            )
            rdma.start()
            sends.append(rdma)

        for rdma in sends[:10]:
            rdma.wait_send()

        for j in range(1, 4):
            gs = r * 4 + jnp.mod(y + j, 4)
            pltpu.make_async_remote_copy(
                src_ref=ag_ref.at[pl.ds(gs, 1)], dst_ref=ag_ref.at[pl.ds(gs, 1)],
                send_sem=agA_ssem.at[j], recv_sem=agA_rsem.at[j],
                device_id=(my,), device_id_type=pl.DeviceIdType.MESH,
            ).wait_recv()

        ag2_sends = []
        for kk in range(1, 8):
            rdma = pltpu.make_async_remote_copy(
                src_ref=ag_ref.at[pl.ds(r * 4, 4)],
                dst_ref=ag_ref.at[pl.ds(r * 4, 4)],
                send_sem=agB_ssem.at[kk],
                recv_sem=agB_rsem.at[8 - kk],
                device_id=(p_grid[kk - 1],),
                device_id_type=pl.DeviceIdType.MESH,
            )
            rdma.start()
            ag2_sends.append(rdma)

        for rdma in sends[10:]:
            rdma.wait_send()

        for j in range(1, 8):
            rs = jnp.mod(r + j, 8)
            pltpu.make_async_remote_copy(
                src_ref=ag_ref.at[pl.ds(rs * 4, 4)],
                dst_ref=ag_ref.at[pl.ds(rs * 4, 4)],
                send_sem=agB_ssem.at[j], recv_sem=agB_rsem.at[j],
                device_id=(my,), device_id_type=pl.DeviceIdType.MESH,
            ).wait_recv()

        out_ref[:, :] = ag_ref[:, :, :].astype(jnp.float32).reshape(m, n)

        for rdma in ag2_sends:
            rdma.wait_send()

    return pl.pallas_call(
        body,
        out_shape=jax.ShapeDtypeStruct((m, n), jnp.float32),
        in_specs=[
            pl.BlockSpec(memory_space=pltpu.VMEM),
            pl.BlockSpec(memory_space=pltpu.VMEM),
        ],
        out_specs=pl.BlockSpec(memory_space=pltpu.VMEM),
        scratch_shapes=[
            pltpu.VMEM((m, k), jnp.bfloat16),
            pltpu.VMEM((8, pc, k), jnp.bfloat16),
            pltpu.VMEM((pc, k), jnp.bfloat16),
            pltpu.VMEM((4, ch, k), jnp.bfloat16),
            pltpu.VMEM((N_DEV, ch, n), jnp.bfloat16),
            pltpu.SemaphoreType.DMA((8,)),
            pltpu.SemaphoreType.DMA((8,)),
            pltpu.SemaphoreType.DMA((4,)),
            pltpu.SemaphoreType.DMA((4,)),
            pltpu.SemaphoreType.DMA((4,)),
            pltpu.SemaphoreType.DMA((4,)),
            pltpu.SemaphoreType.DMA((8,)),
            pltpu.SemaphoreType.DMA((8,)),
        ],
        compiler_params=pltpu.CompilerParams(collective_id=0),
    )(t, W)
